# baseline (speedup 1.0000x reference)
import functools

import jax
import jax.numpy as jnp
from jax import lax
from jax.experimental import pallas as pl
from jax.experimental.pallas import tpu as pltpu
from jax.experimental.pallas import tpu_sc as plsc

N_NODES = 10000
N_EDGES = 320000
HIDDEN = 128

NC = 2
NS = 16
NW = NC * NS

CHUNK = 128
N_CHUNKS = N_EDGES // CHUNK
CH_PER_TILE = N_CHUNKS // NW
N_EXTRA = N_CHUNKS - CH_PER_TILE * NW
ROWS_PER_TILE = 624
REM_ROWS = N_NODES - NS * ROWS_PER_TILE


def _sc_segment_sum(edge_index, edge_attr):
    mesh = plsc.VectorSubcoreMesh(core_axis_name="c", subcore_axis_name="s")

    @functools.partial(
        pl.kernel,
        out_type=[
            jax.ShapeDtypeStruct((N_NODES, HIDDEN), jnp.float32),
            jax.ShapeDtypeStruct((N_NODES, HIDDEN), jnp.float32),
        ],
        mesh=mesh,
        scratch_types=[
            pltpu.VMEM((1, CHUNK), jnp.int32),
            pltpu.VMEM((1, CHUNK), jnp.int32),
            pltpu.VMEM((CHUNK, HIDDEN), jnp.float32),
            pltpu.VMEM((CHUNK, HIDDEN), jnp.float32),
            pltpu.VMEM_SHARED((N_NODES, HIDDEN), jnp.float32),
            pltpu.SemaphoreType.DMA,
            pltpu.SemaphoreType.DMA,
            pltpu.SemaphoreType.DMA,
            pltpu.SemaphoreType.DMA,
        ],
    )
    def seg_sum(idx_hbm, edges_hbm, out0_hbm, out1_hbm,
                idx_v0, idx_v1, rows_v0, rows_v1, acc_sh,
                fsem0, fsem1, ssem0, ssem1):
        cid = lax.axis_index("c")
        sid = lax.axis_index("s")
        wid = sid * NC + cid

        zvec = jnp.zeros((16,), jnp.float32)

        def zero_row(r, carry):
            for c in range(HIDDEN // 16):
                rows_v0[r, pl.ds(c * 16, 16)] = zvec
            return carry

        lax.fori_loop(0, CHUNK, zero_row, 0)
        rbase = sid * ROWS_PER_TILE
        for t in range(ROWS_PER_TILE // CHUNK):
            pltpu.sync_copy(rows_v0, acc_sh.at[pl.ds(rbase + t * CHUNK, CHUNK)])
        tail = ROWS_PER_TILE % CHUNK
        pltpu.sync_copy(
            rows_v0.at[pl.ds(0, tail)],
            acc_sh.at[pl.ds(rbase + ROWS_PER_TILE - tail, tail)],
        )

        @pl.when(sid == 0)
        def _():
            pltpu.sync_copy(
                rows_v0.at[pl.ds(0, REM_ROWS)],
                acc_sh.at[pl.ds(NS * ROWS_PER_TILE, REM_ROWS)],
            )

        idx_b = [idx_v0, idx_v1]
        rows_b = [rows_v0, rows_v1]
        fsem = [fsem0, fsem1]
        ssem = [ssem0, ssem1]

        def fetch(j, b):
            c = j * NW + wid
            pltpu.async_copy(
                idx_hbm.at[0, pl.ds(c * CHUNK, CHUNK)], idx_b[b].at[0], fsem[b])
            pltpu.async_copy(
                edges_hbm.at[pl.ds(c * CHUNK, CHUNK)], rows_b[b], fsem[b])

        def wait_fetch(b):
            pltpu.make_async_copy(
                idx_hbm.at[0, pl.ds(0, CHUNK)], idx_b[b].at[0], fsem[b]).wait()
            pltpu.make_async_copy(
                edges_hbm.at[pl.ds(0, CHUNK)], rows_b[b], fsem[b]).wait()

        def scat(b):
            pltpu.async_copy(rows_b[b], acc_sh.at[idx_b[b].at[0]], ssem[b],
                             add=True)

        def wait_scat(b):
            pltpu.make_async_copy(
                rows_b[b], acc_sh.at[idx_b[b].at[0]], ssem[b]).wait()

        fetch(0, 0)
        fetch(1, 1)
        plsc.subcore_barrier()

        def pair(g, carry):
            wait_fetch(0)
            scat(0)
            wait_fetch(1)
            scat(1)
            wait_scat(0)
            fetch(2 * g + 2, 0)
            wait_scat(1)
            fetch(2 * g + 3, 1)
            return carry

        lax.fori_loop(0, CH_PER_TILE // 2 - 1, pair, 0)
        wait_fetch(0)
        scat(0)
        wait_fetch(1)
        scat(1)
        wait_scat(0)
        wait_scat(1)

        @pl.when(wid < N_EXTRA)
        def _():
            c = CH_PER_TILE * NW + wid
            pltpu.sync_copy(idx_hbm.at[0, pl.ds(c * CHUNK, CHUNK)],
                            idx_v0.at[0])
            pltpu.sync_copy(edges_hbm.at[pl.ds(c * CHUNK, CHUNK)], rows_v0)
            pltpu.sync_copy(rows_v0, acc_sh.at[idx_v0.at[0]], add=True)

        plsc.subcore_barrier()

        @pl.when(cid == 0)
        def _():
            pltpu.sync_copy(
                acc_sh.at[pl.ds(sid * ROWS_PER_TILE, ROWS_PER_TILE)],
                out0_hbm.at[pl.ds(sid * ROWS_PER_TILE, ROWS_PER_TILE)],
            )

            @pl.when(sid == 0)
            def _():
                pltpu.sync_copy(
                    acc_sh.at[pl.ds(NS * ROWS_PER_TILE, REM_ROWS)],
                    out0_hbm.at[pl.ds(NS * ROWS_PER_TILE, REM_ROWS)],
                )

        @pl.when(cid == 1)
        def _():
            pltpu.sync_copy(
                acc_sh.at[pl.ds(sid * ROWS_PER_TILE, ROWS_PER_TILE)],
                out1_hbm.at[pl.ds(sid * ROWS_PER_TILE, ROWS_PER_TILE)],
            )

            @pl.when(sid == 0)
            def _():
                pltpu.sync_copy(
                    acc_sh.at[pl.ds(NS * ROWS_PER_TILE, REM_ROWS)],
                    out1_hbm.at[pl.ds(NS * ROWS_PER_TILE, REM_ROWS)],
                )

    return seg_sum(edge_index, edge_attr)


ROW_BLK = 1000


def _mlp_body(x_ref, p0_ref, p1_ref, w1a_ref, w1b_ref, b1_ref, w2_ref, b2_ref, o_ref):
    xb = x_ref[...]
    s = p0_ref[...] + p1_ref[...]
    h = jnp.dot(xb, w1a_ref[...], preferred_element_type=jnp.float32)
    h = h + jnp.dot(s, w1b_ref[...], preferred_element_type=jnp.float32)
    h = jnp.maximum(h + b1_ref[...], 0.0)
    o = jnp.dot(h, w2_ref[...], preferred_element_type=jnp.float32)
    o_ref[...] = o + b2_ref[...] + xb


def _tc_mlp(x, p0, p1, w1a, w1b, b1, w2, b2):
    grid = (N_NODES // ROW_BLK,)
    blk = lambda i: (i, 0)
    fixed = lambda i: (0, 0)
    return pl.pallas_call(
        _mlp_body,
        grid=grid,
        in_specs=[
            pl.BlockSpec((ROW_BLK, HIDDEN), blk),
            pl.BlockSpec((ROW_BLK, HIDDEN), blk),
            pl.BlockSpec((ROW_BLK, HIDDEN), blk),
            pl.BlockSpec((HIDDEN, HIDDEN), fixed),
            pl.BlockSpec((HIDDEN, HIDDEN), fixed),
            pl.BlockSpec((1, HIDDEN), fixed),
            pl.BlockSpec((HIDDEN, HIDDEN), fixed),
            pl.BlockSpec((1, HIDDEN), fixed),
        ],
        out_specs=pl.BlockSpec((ROW_BLK, HIDDEN), blk),
        out_shape=jax.ShapeDtypeStruct((N_NODES, HIDDEN), jnp.float32),
    )(x, p0, p1, w1a, w1b, b1, w2, b2)


def kernel(x, edge_index, edge_attr, u, batch, W1, b1, W2, b2):
    p0, p1 = _sc_segment_sum(edge_index.astype(jnp.int32), edge_attr)
    return _tc_mlp(
        x, p0, p1,
        W1[:HIDDEN], W1[HIDDEN:],
        b1.reshape(1, HIDDEN),
        W2, b2.reshape(1, HIDDEN),
    )


# speedup vs baseline: 1.3765x; 1.3765x over previous
import functools

import jax
import jax.numpy as jnp
from jax import lax
from jax.experimental import pallas as pl
from jax.experimental.pallas import tpu as pltpu
from jax.experimental.pallas import tpu_sc as plsc

N_NODES = 10000
N_EDGES = 320000
HIDDEN = 128

NC = 2
NS = 16
NW = NC * NS

EDGES_PER_TILE = N_EDGES // NW
CHUNK = 80
N_CH = EDGES_PER_TILE // CHUNK
REPACK = 10240
ROWS_PER_TILE = 624
REM_ROWS = N_NODES - NS * ROWS_PER_TILE


def _sc_segment_sum(edge_index, edge_attr):
    mesh = plsc.VectorSubcoreMesh(core_axis_name="c", subcore_axis_name="s")

    @functools.partial(
        pl.kernel,
        out_type=[
            jax.ShapeDtypeStruct((N_NODES, HIDDEN), jnp.float32),
            jax.ShapeDtypeStruct((N_NODES, HIDDEN), jnp.float32),
            jax.ShapeDtypeStruct((N_EDGES,), jnp.int32),
        ],
        mesh=mesh,
        scratch_types=[
            pltpu.VMEM((REPACK,), jnp.int32),
            pltpu.VMEM((CHUNK,), jnp.int32),
            pltpu.VMEM((CHUNK,), jnp.int32),
            pltpu.VMEM((CHUNK,), jnp.int32),
            pltpu.VMEM((CHUNK, HIDDEN), jnp.float32),
            pltpu.VMEM((CHUNK, HIDDEN), jnp.float32),
            pltpu.VMEM((CHUNK, HIDDEN), jnp.float32),
            pltpu.VMEM_SHARED((N_NODES, HIDDEN), jnp.float32),
            pltpu.SemaphoreType.DMA,
            pltpu.SemaphoreType.DMA,
            pltpu.SemaphoreType.DMA,
            pltpu.SemaphoreType.DMA,
            pltpu.SemaphoreType.DMA,
            pltpu.SemaphoreType.DMA,
        ],
    )
    def seg_sum(ei_hbm, edges_hbm, out0_hbm, out1_hbm, idx_hbm,
                rep_v, idx_v0, idx_v1, idx_v2, rows_v0, rows_v1, rows_v2,
                acc_sh, fsem0, fsem1, fsem2, ssem0, ssem1, ssem2):
        cid = lax.axis_index("c")
        sid = lax.axis_index("s")
        wid = sid * NC + cid
        base = wid * EDGES_PER_TILE

        a0 = pl.multiple_of(lax.div(base, 128) * 128, 128)
        r0 = pl.multiple_of(base - a0, 8)
        pltpu.sync_copy(ei_hbm.at[0, pl.ds(a0, REPACK)], rep_v)
        pltpu.sync_copy(rep_v.at[pl.ds(r0, EDGES_PER_TILE)],
                        idx_hbm.at[pl.ds(base, EDGES_PER_TILE)])

        zvec = jnp.zeros((16,), jnp.float32)

        def zero_row(r, carry):
            for c in range(HIDDEN // 16):
                rows_v0[r, pl.ds(c * 16, 16)] = zvec
            return carry

        lax.fori_loop(0, CHUNK, zero_row, 0)
        rbase = sid * ROWS_PER_TILE
        for t in range(ROWS_PER_TILE // CHUNK):
            pltpu.sync_copy(rows_v0, acc_sh.at[pl.ds(rbase + t * CHUNK, CHUNK)])
        tail = ROWS_PER_TILE % CHUNK
        pltpu.sync_copy(
            rows_v0.at[pl.ds(0, tail)],
            acc_sh.at[pl.ds(rbase + ROWS_PER_TILE - tail, tail)],
        )

        @pl.when(sid == 0)
        def _():
            pltpu.sync_copy(
                rows_v0.at[pl.ds(0, REM_ROWS)],
                acc_sh.at[pl.ds(NS * ROWS_PER_TILE, REM_ROWS)],
            )

        idx_b = [idx_v0, idx_v1, idx_v2]
        rows_b = [rows_v0, rows_v1, rows_v2]
        fsem = [fsem0, fsem1, fsem2]
        ssem = [ssem0, ssem1, ssem2]

        def fetch(j, b):
            pltpu.async_copy(
                idx_hbm.at[pl.ds(base + j * CHUNK, CHUNK)], idx_b[b], fsem[b])
            pltpu.async_copy(
                edges_hbm.at[pl.ds(base + j * CHUNK, CHUNK)], rows_b[b], fsem[b])

        def wait_fetch(b):
            pltpu.make_async_copy(
                idx_hbm.at[pl.ds(0, CHUNK)], idx_b[b], fsem[b]).wait()
            pltpu.make_async_copy(
                edges_hbm.at[pl.ds(0, CHUNK)], rows_b[b], fsem[b]).wait()

        def scat(b):
            pltpu.async_copy(rows_b[b], acc_sh.at[idx_b[b]], ssem[b], add=True)

        def wait_scat(b):
            pltpu.make_async_copy(
                rows_b[b], acc_sh.at[idx_b[b]], ssem[b]).wait()

        fetch(0, 0)
        fetch(1, 1)
        plsc.subcore_barrier()

        wait_fetch(0)
        scat(0)
        fetch(2, 2)
        wait_fetch(1)
        scat(1)
        wait_scat(0)
        fetch(3, 0)

        def group(t, carry):
            j = 3 * t + 2
            for k, (b, bp) in enumerate(((2, 1), (0, 2), (1, 0))):
                wait_fetch(b)
                scat(b)
                wait_scat(bp)
                fetch(j + k + 2, bp)
            return carry

        lax.fori_loop(0, (N_CH - 5) // 3, group, 0)
        wait_fetch(2)
        scat(2)
        wait_scat(1)
        fetch(124, 1)
        wait_fetch(0)
        scat(0)
        wait_scat(2)
        wait_fetch(1)
        scat(1)
        wait_scat(0)
        wait_scat(1)
        plsc.subcore_barrier()

        @pl.when(cid == 0)
        def _():
            pltpu.sync_copy(
                acc_sh.at[pl.ds(sid * ROWS_PER_TILE, ROWS_PER_TILE)],
                out0_hbm.at[pl.ds(sid * ROWS_PER_TILE, ROWS_PER_TILE)],
            )

            @pl.when(sid == 0)
            def _():
                pltpu.sync_copy(
                    acc_sh.at[pl.ds(NS * ROWS_PER_TILE, REM_ROWS)],
                    out0_hbm.at[pl.ds(NS * ROWS_PER_TILE, REM_ROWS)],
                )

        @pl.when(cid == 1)
        def _():
            pltpu.sync_copy(
                acc_sh.at[pl.ds(sid * ROWS_PER_TILE, ROWS_PER_TILE)],
                out1_hbm.at[pl.ds(sid * ROWS_PER_TILE, ROWS_PER_TILE)],
            )

            @pl.when(sid == 0)
            def _():
                pltpu.sync_copy(
                    acc_sh.at[pl.ds(NS * ROWS_PER_TILE, REM_ROWS)],
                    out1_hbm.at[pl.ds(NS * ROWS_PER_TILE, REM_ROWS)],
                )

    return seg_sum(edge_index, edge_attr)


ROW_BLK = 1000


def _mlp_body(x_ref, p0_ref, p1_ref, w1a_ref, w1b_ref, b1_ref, w2_ref, b2_ref, o_ref):
    xb = x_ref[...]
    s = p0_ref[...] + p1_ref[...]
    h = jnp.dot(xb, w1a_ref[...], preferred_element_type=jnp.float32)
    h = h + jnp.dot(s, w1b_ref[...], preferred_element_type=jnp.float32)
    h = jnp.maximum(h + b1_ref[...], 0.0)
    o = jnp.dot(h, w2_ref[...], preferred_element_type=jnp.float32)
    o_ref[...] = o + b2_ref[...] + xb


def _tc_mlp(x, p0, p1, w1a, w1b, b1, w2, b2):
    grid = (N_NODES // ROW_BLK,)
    blk = lambda i: (i, 0)
    fixed = lambda i: (0, 0)
    return pl.pallas_call(
        _mlp_body,
        grid=grid,
        in_specs=[
            pl.BlockSpec((ROW_BLK, HIDDEN), blk),
            pl.BlockSpec((ROW_BLK, HIDDEN), blk),
            pl.BlockSpec((ROW_BLK, HIDDEN), blk),
            pl.BlockSpec((HIDDEN, HIDDEN), fixed),
            pl.BlockSpec((HIDDEN, HIDDEN), fixed),
            pl.BlockSpec((1, HIDDEN), fixed),
            pl.BlockSpec((HIDDEN, HIDDEN), fixed),
            pl.BlockSpec((1, HIDDEN), fixed),
        ],
        out_specs=pl.BlockSpec((ROW_BLK, HIDDEN), blk),
        out_shape=jax.ShapeDtypeStruct((N_NODES, HIDDEN), jnp.float32),
    )(x, p0, p1, w1a, w1b, b1, w2, b2)


def kernel(x, edge_index, edge_attr, u, batch, W1, b1, W2, b2):
    p0, p1, _ = _sc_segment_sum(edge_index.astype(jnp.int32), edge_attr)
    return _tc_mlp(
        x, p0, p1,
        W1[:HIDDEN], W1[HIDDEN:],
        b1.reshape(1, HIDDEN),
        W2, b2.reshape(1, HIDDEN),
    )
